# baseline (speedup 1.0000x reference)
import functools

import jax
import jax.numpy as jnp
import numpy as np
from jax import lax
from jax.experimental import pallas as pl
from jax.experimental.pallas import tpu as pltpu
from jax.experimental.pallas import tpu_sc as plsc

ROWS, COLS = 4096, 8192
LANES = 16
R = 2
CHUNKS = COLS // LANES
CPB = 8
BLOCKS = CHUNKS // CPB

_info = plsc.get_sparse_core_info()
NC, NS = _info.num_cores, _info.num_subcores
NW = NC * NS
ROWS_PER_W = ROWS // NW
GROUPS = ROWS_PER_W // R
NSLOT = 3

_SHIFTS = (1, 2, 4, 8)
_IDX = tuple(
    np.maximum(np.arange(16) - k, 0).astype(np.int32).reshape(16, 1)
    for k in _SHIFTS)
_ZMASK = tuple(
    (np.arange(16) >= k).astype(np.float32) for k in _SHIFTS)

_GD = lax.GatherDimensionNumbers(
    offset_dims=(), collapsed_slice_dims=(0,), start_index_map=(0,))


def _body(x_hbm, m_hbm, out_hbm,
          xb0, xb1, xb2, mb0, mb1, mb2,
          sin0, sin1, sin2, sout0, sout1, sout2):
    wid = lax.axis_index("s") * NC + lax.axis_index("c")
    base = wid * ROWS_PER_W
    xbs = (xb0, xb1, xb2)
    mbs = (mb0, mb1, mb2)
    sins = (sin0, sin1, sin2)
    souts = (sout0, sout1, sout2)

    lane = lax.iota(jnp.int32, LANES)
    half0 = (lane // 8) * 8
    idxs = [jnp.maximum(lane - k, half0).reshape(LANES, 1) for k in (1, 2, 4)]
    zmasks = [((lane % 8) >= k).astype(jnp.float32) for k in (1, 2, 4)]
    himask = (lane >= 8).astype(jnp.float32)

    def logscan(v):
        s = v
        for t in range(4):
            sh = lax.gather(s, idxs[t], _GD, (1,),
                            mode=lax.GatherScatterMode.PROMISE_IN_BOUNDS)
            s = s + sh * zmasks[t]
        return s

    def start_load(g, slot):
        row0 = base + g * R
        pltpu.async_copy(x_hbm.at[pl.ds(row0, R)], xbs[slot], sins[slot])
        pltpu.async_copy(m_hbm.at[pl.ds(row0, R)], mbs[slot], sins[slot])

    def wait_load(slot):
        pltpu.make_async_copy(x_hbm.at[pl.ds(0, R)], xbs[slot], sins[slot]).wait()
        pltpu.make_async_copy(m_hbm.at[pl.ds(0, R)], mbs[slot], sins[slot]).wait()

    def start_store(g, slot):
        row0 = base + g * R
        pltpu.async_copy(xbs[slot], out_hbm.at[pl.ds(row0, R)], souts[slot])

    def wait_store(slot):
        pltpu.make_async_copy(xbs[slot], out_hbm.at[pl.ds(0, R)], souts[slot]).wait()

    def compute(slot):
        xb, mb = xbs[slot], mbs[slot]
        units = [(c, r) for c in range(CPB) for r in range(R)]

        def block(j, carries):
            carries = list(carries)
            offs = {(c, r): (j * CPB + c) * LANES for c, r in units}
            s = {u: xb[u[1], pl.ds(offs[u], LANES)]
                 * mb[u[1], pl.ds(offs[u], LANES)] for u in units}
            for t in range(3):
                sh = {u: lax.gather(
                    s[u], idxs[t], _GD, (1,),
                    mode=lax.GatherScatterMode.PROMISE_IN_BOUNDS)
                    for u in units}
                sh = {u: sh[u] * zmasks[t] for u in units}
                s = {u: s[u] + sh[u] for u in units}
            lo = {u: s[u][7] for u in units}
            hi = {u: s[u][LANES - 1] for u in units}
            s = {u: s[u] + lo[u] * himask for u in units}
            for c in range(CPB):
                for r in range(R):
                    xb[r, pl.ds(offs[(c, r)], LANES)] = s[(c, r)] + carries[r]
                    carries[r] = carries[r] + (lo[(c, r)] + hi[(c, r)])
            return tuple(carries)

        lax.fori_loop(0, BLOCKS, block, (jnp.float32(0),) * R, unroll=False)

    def step(g, slot, *, traced):
        when = pl.when if traced else (lambda p: (lambda f: f() if p else None))
        nxt = (slot + 1) % NSLOT

        @when(g >= 2)
        def _w():
            wait_store(nxt)

        @when(g < GROUPS - 1)
        def _l():
            start_load(g + 1, nxt)

        wait_load(slot)
        compute(slot)
        start_store(g, slot)

    start_load(0, 0)

    def ring(i, carry):
        for k in range(NSLOT):
            step(i * NSLOT + k, k, traced=True)
        return carry

    main_iters = GROUPS // NSLOT
    lax.fori_loop(0, main_iters, ring, 0, unroll=False)
    for g in range(main_iters * NSLOT, GROUPS):
        step(g, g % NSLOT, traced=False)
    wait_store((GROUPS - 2) % NSLOT)
    wait_store((GROUPS - 1) % NSLOT)


@jax.jit
def _masked_cumsum(x, mf):
    mesh = plsc.VectorSubcoreMesh(core_axis_name="c", subcore_axis_name="s")
    return pl.kernel(
        _body,
        out_type=jax.ShapeDtypeStruct((ROWS, COLS), jnp.float32),
        mesh=mesh,
        scratch_types=[
            pltpu.VMEM((R, COLS), jnp.float32),
            pltpu.VMEM((R, COLS), jnp.float32),
            pltpu.VMEM((R, COLS), jnp.float32),
            pltpu.VMEM((R, COLS), jnp.float32),
            pltpu.VMEM((R, COLS), jnp.float32),
            pltpu.VMEM((R, COLS), jnp.float32),
            pltpu.SemaphoreType.DMA,
            pltpu.SemaphoreType.DMA,
            pltpu.SemaphoreType.DMA,
            pltpu.SemaphoreType.DMA,
            pltpu.SemaphoreType.DMA,
            pltpu.SemaphoreType.DMA,
        ],
        compiler_params=pltpu.CompilerParams(needs_layout_passes=False),
    )(x, mf)


def kernel(x, mask):
    return _masked_cumsum(x, mask.astype(jnp.float32))


# speedup vs baseline: 1.0414x; 1.0414x over previous
import functools

import jax
import jax.numpy as jnp
import numpy as np
from jax import lax
from jax.experimental import pallas as pl
from jax.experimental.pallas import tpu as pltpu
from jax.experimental.pallas import tpu_sc as plsc

ROWS, COLS = 4096, 8192
LANES = 16
R = 2
CHUNKS = COLS // LANES
CPB = 4
BLOCKS = CHUNKS // CPB

_info = plsc.get_sparse_core_info()
NC, NS = _info.num_cores, _info.num_subcores
NW = NC * NS
ROWS_PER_W = ROWS // NW
GROUPS = ROWS_PER_W // R
NSLOT = 3

_GD = lax.GatherDimensionNumbers(
    offset_dims=(), collapsed_slice_dims=(0,), start_index_map=(0,))


def _body(x_hbm, m_hbm, out_hbm,
          xb0, xb1, xb2, mb0, mb1, mb2,
          sin0, sin1, sin2, sout0, sout1, sout2):
    wid = lax.axis_index("s") * NC + lax.axis_index("c")
    base = wid * ROWS_PER_W
    xbs = (xb0, xb1, xb2)
    mbs = (mb0, mb1, mb2)
    sins = (sin0, sin1, sin2)
    souts = (sout0, sout1, sout2)

    lane = lax.iota(jnp.int32, LANES)
    half0 = (lane // 8) * 8
    idxs = [jnp.maximum(lane - k, half0).reshape(LANES, 1) for k in (1, 2, 4)]
    zmasks = [((lane % 8) >= k).astype(jnp.float32) for k in (1, 2, 4)]
    himask = (lane >= 8).astype(jnp.float32)

    def start_load(g, slot):
        row0 = base + g * R
        pltpu.async_copy(x_hbm.at[pl.ds(row0, R)], xbs[slot], sins[slot])
        pltpu.async_copy(m_hbm.at[pl.ds(row0, R)], mbs[slot], sins[slot])

    def wait_load(slot):
        pltpu.make_async_copy(x_hbm.at[pl.ds(0, R)], xbs[slot], sins[slot]).wait()
        pltpu.make_async_copy(m_hbm.at[pl.ds(0, R)], mbs[slot], sins[slot]).wait()

    def start_store(g, slot):
        row0 = base + g * R
        pltpu.async_copy(xbs[slot], out_hbm.at[pl.ds(row0, R)], souts[slot])

    def wait_store(slot):
        pltpu.make_async_copy(xbs[slot], out_hbm.at[pl.ds(0, R)], souts[slot]).wait()

    def compute(slot):
        xb, mb = xbs[slot], mbs[slot]
        units = [(c, r) for c in range(CPB) for r in range(R)]

        def block(j, carries):
            carries = list(carries)
            offs = {(c, r): (j * CPB + c) * LANES for c, r in units}
            s = {u: xb[u[1], pl.ds(offs[u], LANES)]
                 * mb[u[1], pl.ds(offs[u], LANES)] for u in units}
            for t in range(3):
                sh = {u: lax.gather(
                    s[u], idxs[t], _GD, (1,),
                    mode=lax.GatherScatterMode.PROMISE_IN_BOUNDS)
                    for u in units}
                sh = {u: sh[u] * zmasks[t] for u in units}
                s = {u: s[u] + sh[u] for u in units}
            lo = {u: s[u][7] for u in units}
            hi = {u: s[u][LANES - 1] for u in units}
            s = {u: s[u] + lo[u] * himask for u in units}
            for c in range(CPB):
                for r in range(R):
                    xb[r, pl.ds(offs[(c, r)], LANES)] = s[(c, r)] + carries[r]
                    carries[r] = carries[r] + (lo[(c, r)] + hi[(c, r)])
            return tuple(carries)

        lax.fori_loop(0, BLOCKS, block, (jnp.float32(0),) * R, unroll=False)

    def step(g, slot, *, traced):
        when = pl.when if traced else (lambda p: (lambda f: f() if p else None))
        nxt = (slot + 1) % NSLOT

        @when(g >= 2)
        def _w():
            wait_store(nxt)

        @when(g < GROUPS - 1)
        def _l():
            start_load(g + 1, nxt)

        wait_load(slot)
        compute(slot)
        start_store(g, slot)

    start_load(0, 0)

    def ring(i, carry):
        for k in range(NSLOT):
            step(i * NSLOT + k, k, traced=True)
        return carry

    main_iters = GROUPS // NSLOT
    lax.fori_loop(0, main_iters, ring, 0, unroll=False)
    for g in range(main_iters * NSLOT, GROUPS):
        step(g, g % NSLOT, traced=False)
    wait_store((GROUPS - 2) % NSLOT)
    wait_store((GROUPS - 1) % NSLOT)


@jax.jit
def _masked_cumsum(x, mf):
    mesh = plsc.VectorSubcoreMesh(core_axis_name="c", subcore_axis_name="s")
    return pl.kernel(
        _body,
        out_type=jax.ShapeDtypeStruct((ROWS, COLS), jnp.float32),
        mesh=mesh,
        scratch_types=[
            pltpu.VMEM((R, COLS), jnp.float32),
            pltpu.VMEM((R, COLS), jnp.float32),
            pltpu.VMEM((R, COLS), jnp.float32),
            pltpu.VMEM((R, COLS), jnp.float32),
            pltpu.VMEM((R, COLS), jnp.float32),
            pltpu.VMEM((R, COLS), jnp.float32),
            pltpu.SemaphoreType.DMA,
            pltpu.SemaphoreType.DMA,
            pltpu.SemaphoreType.DMA,
            pltpu.SemaphoreType.DMA,
            pltpu.SemaphoreType.DMA,
            pltpu.SemaphoreType.DMA,
        ],
        compiler_params=pltpu.CompilerParams(needs_layout_passes=False),
    )(x, mf)


def kernel(x, mask):
    return _masked_cumsum(x, mask.astype(jnp.float32))
